# baseline (speedup 1.0000x reference)
import functools

import jax
import jax.numpy as jnp
from jax import lax
from jax.experimental import pallas as pl
from jax.experimental.pallas import tpu as pltpu
from jax.experimental.pallas import tpu_sc as plsc

NC = 2
NS = 16
L = 16
NW = NC * NS

E = 5888
UNROLL = 8
TPW = 4

_HI = jnp.int32(-65536)
_RND = jnp.int32(32768)


@functools.partial(jax.jit, static_argnames=("n_chunks", "shift"))
def _sc_spmm(x2, cr, vv, bias, n_chunks, shift):
    T, n_in = x2.shape
    n_out = bias.shape[0]
    half = n_in // 2
    cmask = (1 << shift) - 1
    mesh = plsc.VectorSubcoreMesh(
        core_axis_name="c", subcore_axis_name="s", num_cores=NC, num_subcores=NS
    )

    @functools.partial(
        pl.kernel,
        out_type=jax.ShapeDtypeStruct((T, n_out), jnp.float32),
        mesh=mesh,
        compiler_params=pltpu.CompilerParams(needs_layout_passes=False),
        scratch_types=(
            [pltpu.VMEM((n_in,), jnp.float32) for _ in range(TPW // 2)]
            + [pltpu.VMEM((half,), jnp.float32)]
            + [pltpu.VMEM((n_out,), jnp.float32) for _ in range(TPW)]
            + [pltpu.VMEM((E,), jnp.int32) for _ in range(4)]
            + [pltpu.SemaphoreType.DMA for _ in range(2)]
        ),
    )
    def body(x_hbm, cr_hbm, vv_hbm, bias_hbm, out_hbm, *scratch):
        x_v = scratch[:TPW // 2]
        stage = scratch[TPW // 2]
        y_v = scratch[TPW // 2 + 1:TPW // 2 + 1 + TPW]
        cr_v = scratch[TPW // 2 + 1 + TPW:TPW // 2 + 3 + TPW]
        vv_v = scratch[TPW // 2 + 3 + TPW:TPW // 2 + 5 + TPW]
        sems = scratch[TPW // 2 + 5 + TPW:]
        wid = lax.axis_index("s") * NC + lax.axis_index("c")
        t0 = wid * TPW

        def start(ci, b):
            pltpu.make_async_copy(cr_hbm.at[ci], cr_v[b], sems[b]).start()
            pltpu.make_async_copy(vv_hbm.at[ci], vv_v[b], sems[b]).start()

        def wait(b):
            pltpu.make_async_copy(cr_hbm.at[0], cr_v[b], sems[b]).wait()
            pltpu.make_async_copy(vv_hbm.at[0], vv_v[b], sems[b]).wait()

        start(0, 0)

        for pp in range(TPW // 2):
            pltpu.sync_copy(x_hbm.at[t0 + 2 * pp], x_v[pp])
            for h in range(2):
                pltpu.sync_copy(
                    x_hbm.at[t0 + 2 * pp + 1, pl.ds(h * half, half)], stage
                )

                @plsc.parallel_loop(0, half, step=L, unroll=8)
                def pack_body(off):
                    xe = plsc.bitcast(
                        x_v[pp][pl.ds(h * half + off, L)], jnp.int32
                    )
                    xo = plsc.bitcast(stage[pl.ds(off, L)], jnp.int32)
                    he = lax.bitwise_and(xe + _RND, _HI)
                    ho = lax.shift_right_logical(xo + _RND, 16)
                    x_v[pp][pl.ds(h * half + off, L)] = plsc.bitcast(
                        lax.bitwise_or(he, ho), jnp.float32
                    )

        for tl in range(TPW):
            pltpu.sync_copy(bias_hbm, y_v[tl])

        def compute(b):
            @plsc.parallel_loop(0, E, step=L, unroll=UNROLL)
            def group_body(off):
                cr16 = cr_v[b][pl.ds(off, L)]
                v16 = plsc.bitcast(vv_v[b][pl.ds(off, L)], jnp.float32)
                c16 = lax.bitwise_and(cr16, cmask)
                r16 = lax.shift_right_logical(cr16, shift)
                for pp in range(TPW // 2):
                    g = plsc.bitcast(plsc.load_gather(x_v[pp], [c16]), jnp.int32)
                    x_ev = plsc.bitcast(lax.bitwise_and(g, _HI), jnp.float32)
                    x_od = plsc.bitcast(lax.shift_left(g, 16), jnp.float32)
                    plsc.addupdate_scatter(y_v[2 * pp], [r16], x_ev * v16)
                    plsc.addupdate_scatter(y_v[2 * pp + 1], [r16], x_od * v16)

        def chunk_pair(ci2, _):
            ci = ci2 * 2

            @pl.when(ci + 1 < n_chunks)
            def _():
                start(ci + 1, 1)

            wait(0)
            compute(0)

            @pl.when(ci + 2 < n_chunks)
            def _():
                start(ci + 2, 0)

            wait(1)
            compute(1)
            return 0

        lax.fori_loop(0, n_chunks // 2, chunk_pair, 0)
        for tl in range(TPW):
            pltpu.sync_copy(y_v[tl], out_hbm.at[t0 + tl])

    return body(x2, cr, vv, bias)


def kernel(x, w_rows, w_cols, w_vals, bias):
    b, s, n_in = x.shape
    t = b * s
    n_out = bias.shape[0]
    shift = (n_in - 1).bit_length()

    nnz = w_rows.shape[0]
    pair = 2 * E
    nnz_pad = ((nnz + pair - 1) // pair) * pair
    pad = nnz_pad - nnz
    n_chunks = nnz_pad // E
    cr = jnp.pad(
        w_rows.astype(jnp.int32) << shift | w_cols, (0, pad)
    ).reshape(n_chunks, E)
    vv = lax.bitcast_convert_type(
        jnp.pad(w_vals, (0, pad)), jnp.int32
    ).reshape(n_chunks, E)
    y = _sc_spmm(x.reshape(t, n_in), cr, vv, bias, n_chunks, shift)
    return y.reshape(b, s, n_out)
